# baseline (speedup 1.0000x reference)
import jax
import jax.numpy as jnp
from jax.experimental import pallas as pl
from jax.experimental.pallas import tpu as pltpu

GROUP = 16
CG = 6
P = 6
NS = 9
H = 224
HO = 222


def _win_coords(flat):
    c, k = flat // 9, flat % 9
    return c, k // 3, k % 3


def _motif_group_kernel(x_ref, o_ref, sh_ref, sel_ref):
    o_ref[...] = jnp.zeros_like(o_ref)

    for c in range(CG):
        for kj in (1, 2):
            sh_ref[(kj - 1) * CG + c, :, 0:H - kj] = x_ref[c, :, kj:H]

    def cref(c, kj):
        return x_ref.at[c] if kj == 0 else sh_ref.at[(kj - 1) * CG + c]

    def wchunk_static(flat, rs, nr):
        c, ki, kj = _win_coords(flat)
        return cref(c, kj)[ki + rs:ki + rs + nr, 0:HO]

    def wchunk_dyn(flat, i):
        c, ki, kj = _win_coords(flat)
        if ki == 0:
            return cref(c, kj)[pl.ds(8 * i, 8), 0:HO]
        sup = cref(c, kj)[pl.ds(8 * i, 16), 0:HO]
        return sup[ki:ki + 8, :]

    tail_mask = jax.lax.broadcasted_iota(jnp.int32, (8, HO), 0) >= 2
    pairs = [(p, q) for p in range(P) for q in range(p + 1, P)]

    for j in range(NS):
        slice_coords = [_win_coords(P * j + t) for t in range(P)]
        chans = sorted({c for (c, _, _) in slice_coords})

        def pa_body(i, accs):
            cache = [wchunk_dyn(P * j + t, i) for t in range(P)]
            out = []
            for idx, (p, q) in enumerate(pairs):
                d = cache[p] - cache[q]
                out.append(accs[idx] + d * d)
            return tuple(out)

        accs = jax.lax.fori_loop(
            0, 27, pa_body,
            tuple(jnp.zeros((8, HO), jnp.float32) for _ in range(15)),
            unroll=1)

        tcache = [wchunk_static(P * j + t, 8 * 27 - 2, 8) for t in range(P)]
        dist = {}
        for idx, (p, q) in enumerate(pairs):
            d = tcache[p] - tcache[q]
            dd = jnp.where(tail_mask, d * d, jnp.float32(0.0))
            dist[(p, q)] = jnp.sqrt(jnp.sum(accs[idx] + dd))

        def D(p, q):
            return dist[(p, q)] if p < q else dist[(q, p)]

        nn = []
        for p in range(P):
            best = jnp.float32(jnp.inf)
            idx = jnp.int32(0)
            for q in range(P):
                if q == p:
                    continue
                better = D(p, q) < best
                best = jnp.where(better, D(p, q), best)
                idx = jnp.where(better, jnp.int32(q), idx)
            nn.append(idx)
        u = nn[0]
        for p in range(1, P):
            u = jnp.maximum(u, nn[p])
        cnt = (nn[0] == u).astype(jnp.float32)
        for p in range(1, P):
            cnt = cnt + (nn[p] == u).astype(jnp.float32)
        factor = cnt * jnp.float32(1.0 / CG)
        w_hot = [(u == t).astype(jnp.float32) for t in range(P)]

        def selc_chunk(i_static_or_dyn, dyn, nr):
            if dyn:
                cache = [wchunk_dyn(P * j + t, i_static_or_dyn)
                         for t in range(P)]
            else:
                cache = [wchunk_static(P * j + t, i_static_or_dyn, nr)
                         for t in range(P)]
            selc = w_hot[0] * cache[0]
            for t in range(1, P):
                selc = selc + w_hot[t] * cache[t]
            return selc * factor

        def selstore_body(i, carry):
            selcf = selc_chunk(i, True, 8)
            for kj in range(3):
                sel_ref[kj, pl.ds(8 * i, 8), kj:kj + HO] = selcf
            return carry

        jax.lax.fori_loop(0, 27, selstore_body, jnp.int32(0), unroll=1)
        selcf_tail = selc_chunk(216, False, 6)
        for kj in range(3):
            sel_ref[kj, 216:222, kj:kj + HO] = selcf_tail

        lane = jax.lax.broadcasted_iota(jnp.int32, (8, H), 1)
        colmask = [(lane >= kj) & (lane < kj + HO) for kj in range(3)]
        rowi = jax.lax.broadcasted_iota(jnp.int32, (8, H), 0)

        def pb_terms(selsups, xloads, rowshift, masks):
            acc = {}
            for t in range(P):
                c, ki, kj = slice_coords[t]
                selsh = rowshift(selsups[kj], ki)
                term = jnp.floor(xloads[c] * selsh)
                m = colmask[kj] if masks is None else masks[t]
                term = jnp.where(m, term, jnp.float32(0.0))
                acc[c] = term if c not in acc else acc[c] + term
            return acc

        def pb_body(i, carry):
            selsups = {}
            for kj in {kj for (_, _, kj) in slice_coords}:
                selsups[kj] = sel_ref[kj, pl.ds(8 * (i - 1), 16), :]
            xloads = {c: x_ref[c, pl.ds(8 * i, 8), :] for c in chans}
            acc = pb_terms(selsups, xloads,
                           lambda sup, ki: sup[8 - ki:16 - ki, :], None)
            for c in chans:
                o_ref[c, pl.ds(8 * i, 8), :] += acc[c]
            return carry

        jax.lax.fori_loop(1, 27, pb_body, jnp.int32(0), unroll=1)

        selsups0 = {kj: sel_ref[kj, 0:16, :]
                    for kj in {kj for (_, _, kj) in slice_coords}}
        xloads0 = {c: x_ref[c, 0:8, :] for c in chans}

        def rowshift0(sup, ki):
            if ki == 0:
                return sup[0:8, :]
            return jnp.concatenate(
                [jnp.zeros((ki, H), jnp.float32), sup[0:8 - ki, :]], axis=0)

        masks0 = [colmask[kj] & (rowi >= ki)
                  for (_, ki, kj) in slice_coords]
        acc0 = pb_terms(selsups0, xloads0, rowshift0, masks0)
        for c in chans:
            o_ref[c, 0:8, :] += acc0[c]

        selsups1 = {kj: sel_ref[kj, 208:224, :]
                    for kj in {kj for (_, _, kj) in slice_coords}}
        xloads1 = {c: x_ref[c, 216:224, :] for c in chans}
        masks1 = [colmask[kj] & (rowi < 6 + ki)
                  for (_, ki, kj) in slice_coords]
        acc1 = pb_terms(selsups1, xloads1,
                        lambda sup, ki: sup[8 - ki:16 - ki, :], masks1)
        for c in chans:
            o_ref[c, 216:224, :] += acc1[c]


def kernel(feature):
    x = feature[0]
    out = pl.pallas_call(
        _motif_group_kernel,
        grid=(GROUP,),
        in_specs=[pl.BlockSpec((CG, H, H), lambda g: (g, 0, 0))],
        out_specs=pl.BlockSpec((CG, H, H), lambda g: (g, 0, 0)),
        out_shape=jax.ShapeDtypeStruct((GROUP * CG, H, H), jnp.float32),
        scratch_shapes=[
            pltpu.VMEM((2 * CG, H, H), jnp.float32),
            pltpu.VMEM((3, H, H), jnp.float32),
        ],
        compiler_params=pltpu.CompilerParams(
            dimension_semantics=("arbitrary",),
        ),
    )(x)
    return out[None]


# speedup vs baseline: 1.5698x; 1.5698x over previous
import jax
import jax.numpy as jnp
from jax.experimental import pallas as pl
from jax.experimental.pallas import tpu as pltpu

GROUP = 16
CG = 6
P = 6
NS = 9
H = 224
HO = 222


def _win_coords(flat):
    c, k = flat // 9, flat % 9
    return c, k // 3, k % 3


def _motif_group_kernel(x_ref, o_ref, sh_ref, sel_ref):
    o_ref[...] = jnp.zeros_like(o_ref)

    for c in range(CG):
        for kj in (1, 2):
            sh_ref[(kj - 1) * CG + c, :, 0:H - kj] = x_ref[c, :, kj:H]

    def cref(c, kj):
        return x_ref.at[c] if kj == 0 else sh_ref.at[(kj - 1) * CG + c]

    def wchunk_static(flat, rs, nr):
        c, ki, kj = _win_coords(flat)
        return cref(c, kj)[ki + rs:ki + rs + nr, 0:HO]

    def wchunk_dyn(flat, i):
        c, ki, kj = _win_coords(flat)
        if ki == 0:
            return cref(c, kj)[pl.ds(8 * i, 8), 0:HO]
        sup = cref(c, kj)[pl.ds(8 * i, 16), 0:HO]
        return sup[ki:ki + 8, :]

    tail_mask = jax.lax.broadcasted_iota(jnp.int32, (8, HO), 0) >= 2
    pairs = [(p, q) for p in range(P) for q in range(p + 1, P)]

    for j in range(NS):
        slice_coords = [_win_coords(P * j + t) for t in range(P)]
        chans = sorted({c for (c, _, _) in slice_coords})

        def pa_body(i, accs):
            cache = [wchunk_dyn(P * j + t, i) for t in range(P)]
            out = []
            for idx, (p, q) in enumerate(pairs):
                d = cache[p] - cache[q]
                out.append(accs[idx] + d * d)
            return tuple(out)

        accs = jax.lax.fori_loop(
            0, 27, pa_body,
            tuple(jnp.zeros((8, HO), jnp.float32) for _ in range(15)),
            unroll=3)

        tcache = [wchunk_static(P * j + t, 8 * 27 - 2, 8) for t in range(P)]
        dist = {}
        for idx, (p, q) in enumerate(pairs):
            d = tcache[p] - tcache[q]
            dd = jnp.where(tail_mask, d * d, jnp.float32(0.0))
            dist[(p, q)] = jnp.sqrt(jnp.sum(accs[idx] + dd))

        def D(p, q):
            return dist[(p, q)] if p < q else dist[(q, p)]

        nn = []
        for p in range(P):
            best = jnp.float32(jnp.inf)
            idx = jnp.int32(0)
            for q in range(P):
                if q == p:
                    continue
                better = D(p, q) < best
                best = jnp.where(better, D(p, q), best)
                idx = jnp.where(better, jnp.int32(q), idx)
            nn.append(idx)
        u = nn[0]
        for p in range(1, P):
            u = jnp.maximum(u, nn[p])
        cnt = (nn[0] == u).astype(jnp.float32)
        for p in range(1, P):
            cnt = cnt + (nn[p] == u).astype(jnp.float32)
        factor = cnt * jnp.float32(1.0 / CG)
        w_hot = [(u == t).astype(jnp.float32) for t in range(P)]

        def selc_chunk(i_static_or_dyn, dyn, nr):
            if dyn:
                cache = [wchunk_dyn(P * j + t, i_static_or_dyn)
                         for t in range(P)]
            else:
                cache = [wchunk_static(P * j + t, i_static_or_dyn, nr)
                         for t in range(P)]
            selc = w_hot[0] * cache[0]
            for t in range(1, P):
                selc = selc + w_hot[t] * cache[t]
            return selc * factor

        def selstore_body(i, carry):
            selcf = selc_chunk(i, True, 8)
            for kj in range(3):
                sel_ref[kj, pl.ds(8 * i, 8), kj:kj + HO] = selcf
            return carry

        jax.lax.fori_loop(0, 27, selstore_body, jnp.int32(0), unroll=3)
        selcf_tail = selc_chunk(216, False, 6)
        for kj in range(3):
            sel_ref[kj, 216:222, kj:kj + HO] = selcf_tail

        lane = jax.lax.broadcasted_iota(jnp.int32, (8, H), 1)
        colmask = [(lane >= kj) & (lane < kj + HO) for kj in range(3)]
        rowi = jax.lax.broadcasted_iota(jnp.int32, (8, H), 0)

        def pb_terms(selsups, xloads, rowshift, masks):
            acc = {}
            for t in range(P):
                c, ki, kj = slice_coords[t]
                selsh = rowshift(selsups[kj], ki)
                term = jnp.floor(xloads[c] * selsh)
                m = colmask[kj] if masks is None else masks[t]
                term = jnp.where(m, term, jnp.float32(0.0))
                acc[c] = term if c not in acc else acc[c] + term
            return acc

        def pb_body(i, carry):
            selsups = {}
            for kj in {kj for (_, _, kj) in slice_coords}:
                selsups[kj] = sel_ref[kj, pl.ds(8 * (i - 1), 16), :]
            xloads = {c: x_ref[c, pl.ds(8 * i, 8), :] for c in chans}
            acc = pb_terms(selsups, xloads,
                           lambda sup, ki: sup[8 - ki:16 - ki, :], None)
            for c in chans:
                o_ref[c, pl.ds(8 * i, 8), :] += acc[c]
            return carry

        jax.lax.fori_loop(1, 27, pb_body, jnp.int32(0), unroll=3)

        selsups0 = {kj: sel_ref[kj, 0:16, :]
                    for kj in {kj for (_, _, kj) in slice_coords}}
        xloads0 = {c: x_ref[c, 0:8, :] for c in chans}

        def rowshift0(sup, ki):
            if ki == 0:
                return sup[0:8, :]
            return jnp.concatenate(
                [jnp.zeros((ki, H), jnp.float32), sup[0:8 - ki, :]], axis=0)

        masks0 = [colmask[kj] & (rowi >= ki)
                  for (_, ki, kj) in slice_coords]
        acc0 = pb_terms(selsups0, xloads0, rowshift0, masks0)
        for c in chans:
            o_ref[c, 0:8, :] += acc0[c]

        selsups1 = {kj: sel_ref[kj, 208:224, :]
                    for kj in {kj for (_, _, kj) in slice_coords}}
        xloads1 = {c: x_ref[c, 216:224, :] for c in chans}
        masks1 = [colmask[kj] & (rowi < 6 + ki)
                  for (_, ki, kj) in slice_coords]
        acc1 = pb_terms(selsups1, xloads1,
                        lambda sup, ki: sup[8 - ki:16 - ki, :], masks1)
        for c in chans:
            o_ref[c, 216:224, :] += acc1[c]


def kernel(feature):
    x = feature[0]
    out = pl.pallas_call(
        _motif_group_kernel,
        grid=(GROUP,),
        in_specs=[pl.BlockSpec((CG, H, H), lambda g: (g, 0, 0))],
        out_specs=pl.BlockSpec((CG, H, H), lambda g: (g, 0, 0)),
        out_shape=jax.ShapeDtypeStruct((GROUP * CG, H, H), jnp.float32),
        scratch_shapes=[
            pltpu.VMEM((2 * CG, H, H), jnp.float32),
            pltpu.VMEM((3, H, H), jnp.float32),
        ],
        compiler_params=pltpu.CompilerParams(
            dimension_semantics=("arbitrary",),
        ),
    )(x)
    return out[None]


# speedup vs baseline: 2.7883x; 1.7762x over previous
import jax
import jax.numpy as jnp
from jax.experimental import pallas as pl
from jax.experimental.pallas import tpu as pltpu

GROUP = 16
CG = 6
P = 6
NS = 9
H = 224
HO = 222

_PASSES = [
    [(0, 1), (0, 2), (0, 3), (0, 4), (0, 5)],
    [(1, 2), (1, 3), (1, 4), (1, 5), (2, 3)],
    [(2, 4), (2, 5), (3, 4), (3, 5), (4, 5)],
]


def _win_coords(flat):
    c, k = flat // 9, flat % 9
    return c, k // 3, k % 3


def _motif_group_kernel(x_ref, o_ref, sh_ref, sel_ref):
    o_ref[...] = jnp.zeros_like(o_ref)

    for c in range(CG):
        for kj in (1, 2):
            sh_ref[(kj - 1) * CG + c, :, 0:H - kj] = x_ref[c, :, kj:H]

    def wchunk(flat, rs, nr=8):
        c, ki, kj = _win_coords(flat)
        if kj == 0:
            return x_ref[c, ki + rs:ki + rs + nr, 0:HO]
        return sh_ref[(kj - 1) * CG + c, ki + rs:ki + rs + nr, 0:HO]

    tail_mask = jax.lax.broadcasted_iota(jnp.int32, (8, HO), 0) >= 2

    for j in range(NS):
        slice_coords = [_win_coords(P * j + t) for t in range(P)]
        chans = sorted({c for (c, _, _) in slice_coords})
        slice_kjs = sorted({kj for (_, _, kj) in slice_coords})

        dist = {}
        for pair_pass in _PASSES:
            accs = {}
            for chunk in range(28):
                rs = 8 * chunk if chunk < 27 else 8 * 27 - 2
                cache = {}
                for (p, q) in pair_pass:
                    for t in (p, q):
                        if t not in cache:
                            cache[t] = wchunk(P * j + t, rs)
                    d = cache[p] - cache[q]
                    dd = d * d
                    if chunk == 27:
                        dd = jnp.where(tail_mask, dd, jnp.float32(0.0))
                    accs[(p, q)] = dd if chunk == 0 else accs[(p, q)] + dd
            for pq in pair_pass:
                dist[pq] = jnp.sqrt(jnp.sum(accs[pq]))

        def D(p, q):
            return dist[(p, q)] if p < q else dist[(q, p)]

        nn = []
        for p in range(P):
            best = jnp.float32(jnp.inf)
            idx = jnp.int32(0)
            for q in range(P):
                if q == p:
                    continue
                better = D(p, q) < best
                best = jnp.where(better, D(p, q), best)
                idx = jnp.where(better, jnp.int32(q), idx)
            nn.append(idx)
        u = nn[0]
        for p in range(1, P):
            u = jnp.maximum(u, nn[p])
        cnt = (nn[0] == u).astype(jnp.float32)
        for p in range(1, P):
            cnt = cnt + (nn[p] == u).astype(jnp.float32)
        factor = cnt * jnp.float32(1.0 / CG)
        w_hot = [(u == t).astype(jnp.float32) for t in range(P)]

        for chunk in range(28):
            rs = 8 * chunk
            nr = 8 if chunk < 27 else 6
            cache = [wchunk(P * j + t, rs, nr) for t in range(P)]
            selc = w_hot[0] * cache[0]
            for t in range(1, P):
                selc = selc + w_hot[t] * cache[t]
            selcf = selc * factor
            for kj in slice_kjs:
                sel_ref[kj, rs:rs + nr, kj:kj + HO] = selcf

        lane = jax.lax.broadcasted_iota(jnp.int32, (8, H), 1)
        colmask = [(lane >= kj) & (lane < kj + HO) for kj in range(3)]
        rowi = jax.lax.broadcasted_iota(jnp.int32, (8, H), 0)

        for chunk in range(28):
            ys = 8 * chunk
            first, last = chunk == 0, chunk == 27
            selsh = {}
            for t in range(P):
                c, ki, kj = slice_coords[t]
                if (ki, kj) in selsh:
                    continue
                if first and ki > 0:
                    part = sel_ref[kj, 0:8 - ki, :]
                    selsh[(ki, kj)] = jnp.concatenate(
                        [jnp.zeros((ki, H), jnp.float32), part], axis=0)
                else:
                    selsh[(ki, kj)] = sel_ref[kj, ys - ki:ys - ki + 8, :]
            xloads = {c: x_ref[c, ys:ys + 8, :] for c in chans}
            acc = {}
            for t in range(P):
                c, ki, kj = slice_coords[t]
                term = jnp.floor(xloads[c] * selsh[(ki, kj)])
                m = colmask[kj]
                if first:
                    m = m & (rowi >= ki)
                if last:
                    m = m & (rowi < 6 + ki)
                term = jnp.where(m, term, jnp.float32(0.0))
                acc[c] = term if c not in acc else acc[c] + term
            for c in chans:
                o_ref[c, ys:ys + 8, :] += acc[c]


def kernel(feature):
    x = feature[0]
    out = pl.pallas_call(
        _motif_group_kernel,
        grid=(GROUP,),
        in_specs=[pl.BlockSpec((CG, H, H), lambda g: (g, 0, 0))],
        out_specs=pl.BlockSpec((CG, H, H), lambda g: (g, 0, 0)),
        out_shape=jax.ShapeDtypeStruct((GROUP * CG, H, H), jnp.float32),
        scratch_shapes=[
            pltpu.VMEM((2 * CG, H, H), jnp.float32),
            pltpu.VMEM((3, H, H), jnp.float32),
        ],
        compiler_params=pltpu.CompilerParams(
            dimension_semantics=("arbitrary",),
        ),
    )(x)
    return out[None]
